# baseline (speedup 1.0000x reference)
import jax
import jax.numpy as jnp
from jax.experimental import pallas as pl
from jax.experimental.pallas import tpu as pltpu

_LN_EPS = 1e-5
_BK = 1000


def _embed_kernel(xt0_ref, xt1_ref, emb_ref, scale_ref, bias_ref, out_ref, gid_ref, acc_ref):
    k = pl.program_id(0)
    nk = pl.num_programs(0)

    @pl.when(k == 0)
    def _iota():
        gid_ref[...] = jax.lax.broadcasted_iota(jnp.int32, gid_ref.shape, 1)

    hb = acc_ref.shape[1] // 2
    eb = emb_ref[...].astype(jnp.bfloat16)
    prod0 = jax.lax.dot_general(
        eb, xt0_ref[...].astype(jnp.bfloat16), (((0,), (0,)), ((), ())),
        preferred_element_type=jnp.float32,
    )
    prod1 = jax.lax.dot_general(
        eb, xt1_ref[...].astype(jnp.bfloat16), (((0,), (0,)), ((), ())),
        preferred_element_type=jnp.float32,
    )

    @pl.when(k == 0)
    def _init():
        acc_ref[:, :hb] = prod0
        acc_ref[:, hb:] = prod1

    @pl.when(k > 0)
    def _accum():
        acc_ref[:, :hb] += prod0
        acc_ref[:, hb:] += prod1

    @pl.when(k == nk - 1)
    def _epilogue():
        h = jax.nn.gelu(acc_ref[...])
        mean = jnp.mean(h, axis=0, keepdims=True)
        var = jnp.mean((h - mean) ** 2, axis=0, keepdims=True)
        res = (h - mean) * jax.lax.rsqrt(var + _LN_EPS) * scale_ref[...] + bias_ref[...]
        out_ref[...] = res.T


def kernel(x, emb, ln_scale, ln_bias):
    B, G = x.shape
    H = emb.shape[1]
    xt = x.T
    scale2 = ln_scale.reshape(H, 1)
    bias2 = ln_bias.reshape(H, 1)
    out = pl.pallas_call(
        _embed_kernel,
        grid=(G // _BK,),
        in_specs=[
            pl.BlockSpec((_BK, B // 2), lambda k: (k, 0)),
            pl.BlockSpec((_BK, B // 2), lambda k: (k, 1)),
            pl.BlockSpec((_BK, H), lambda k: (k, 0)),
            pl.BlockSpec((H, 1), lambda k: (0, 0)),
            pl.BlockSpec((H, 1), lambda k: (0, 0)),
        ],
        out_specs=[
            pl.BlockSpec((B, H), lambda k: (0, 0)),
            pl.BlockSpec((1, G), lambda k: (0, 0)),
        ],
        out_shape=[
            jax.ShapeDtypeStruct((B, H), jnp.float32),
            jax.ShapeDtypeStruct((1, G), jnp.int32),
        ],
        scratch_shapes=[pltpu.VMEM((H, B), jnp.float32)],
        compiler_params=pltpu.CompilerParams(
            dimension_semantics=("arbitrary",)
        ),
    )(xt, xt, emb, scale2, bias2)
    out, gene_idx2 = out
    return (out, gene_idx2.reshape(G))


# speedup vs baseline: 1.0140x; 1.0140x over previous
import jax
import jax.numpy as jnp
from jax.experimental import pallas as pl
from jax.experimental.pallas import tpu as pltpu

_LN_EPS = 1e-5
_BK = 800


def _embed_kernel(xt0_ref, xt1_ref, emb_ref, scale_ref, bias_ref, out_ref, acc_ref):
    k = pl.program_id(0)
    nk = pl.num_programs(0)
    hb = acc_ref.shape[1] // 2
    eb = emb_ref[...].astype(jnp.bfloat16)
    prod0 = jax.lax.dot_general(
        eb, xt0_ref[...].astype(jnp.bfloat16), (((0,), (0,)), ((), ())),
        preferred_element_type=jnp.float32,
    )
    prod1 = jax.lax.dot_general(
        eb, xt1_ref[...].astype(jnp.bfloat16), (((0,), (0,)), ((), ())),
        preferred_element_type=jnp.float32,
    )

    @pl.when(k == 0)
    def _init():
        acc_ref[:, :hb] = prod0
        acc_ref[:, hb:] = prod1

    @pl.when(k > 0)
    def _accum():
        acc_ref[:, :hb] += prod0
        acc_ref[:, hb:] += prod1

    @pl.when(k == nk - 1)
    def _epilogue():
        h = jax.nn.gelu(acc_ref[...])
        mean = jnp.mean(h, axis=0, keepdims=True)
        var = jnp.mean((h - mean) ** 2, axis=0, keepdims=True)
        res = (h - mean) * jax.lax.rsqrt(var + _LN_EPS) * scale_ref[...] + bias_ref[...]
        out_ref[...] = res.T


def kernel(x, emb, ln_scale, ln_bias):
    B, G = x.shape
    H = emb.shape[1]
    xt = x.T
    scale2 = ln_scale.reshape(H, 1)
    bias2 = ln_bias.reshape(H, 1)
    out = pl.pallas_call(
        _embed_kernel,
        grid=(G // _BK,),
        in_specs=[
            pl.BlockSpec((_BK, B // 2), lambda k: (k, 0)),
            pl.BlockSpec((_BK, B // 2), lambda k: (k, 1)),
            pl.BlockSpec((_BK, H), lambda k: (k, 0)),
            pl.BlockSpec((H, 1), lambda k: (0, 0)),
            pl.BlockSpec((H, 1), lambda k: (0, 0)),
        ],
        out_specs=pl.BlockSpec((B, H), lambda k: (0, 0)),
        out_shape=jax.ShapeDtypeStruct((B, H), jnp.float32),
        scratch_shapes=[pltpu.VMEM((H, B), jnp.float32)],
        compiler_params=pltpu.CompilerParams(
            dimension_semantics=("arbitrary",)
        ),
    )(xt, xt, emb, scale2, bias2)
    gene_idx = jnp.arange(G, dtype=jnp.int32)
    return (out, gene_idx)
